# baseline (speedup 1.0000x reference)
import functools

import jax
import jax.numpy as jnp
from jax import lax
from jax.experimental import pallas as pl
from jax.experimental.pallas import tpu as pltpu
from jax.experimental.pallas import tpu_sc as plsc

VOCAB = 1000000
DIM = 64
BATCH = 16384
HIST = 50
PADW = 2 * DIM

VBP = 16384
NPAIR = -(-((VOCAB + VBP - 1) // VBP) // 2) * VBP
NGRID = NPAIR // VBP
TABROWS = 2 * NPAIR

NC = 2
NS = 16
NW = NC * NS
B_PER_W = BATCH // NW
NB = 16
IDX_PER_BLK = NB * HIST
N_BLK = B_PER_W // NB
NCHUNK = DIM // 16
RUNROLL = 25


def _tc_proj_table(emb_t, proj_w, proj_b):
    def proj_kernel(e_ref, w_ref, b_ref, o_ref):
        for half in (0, 1):
            p = lax.dot_general(
                e_ref[:, half * VBP:(half + 1) * VBP].T, w_ref[...],
                (((1,), (1,)), ((), ())),
                preferred_element_type=jnp.float32,
            )
            o_ref[:, half * DIM:(half + 1) * DIM] = p + b_ref[...]

    return pl.pallas_call(
        proj_kernel,
        grid=(NGRID,),
        in_specs=[
            pl.BlockSpec((DIM, 2 * VBP), lambda k: (0, k)),
            pl.BlockSpec((DIM, DIM), lambda k: (0, 0)),
            pl.BlockSpec((1, DIM), lambda k: (0, 0)),
        ],
        out_specs=pl.BlockSpec((VBP, PADW), lambda k: (k, 0)),
        out_shape=jax.ShapeDtypeStruct((NPAIR, PADW), jnp.float32),
        compiler_params=pltpu.CompilerParams(
            fuse_transposed_lhs_in_matmul=True),
    )(emb_t, proj_w, proj_b.reshape(1, DIM))


def _sc_pool_sum(ids_flat, ptab_flat):
    mesh = plsc.VectorSubcoreMesh(core_axis_name="c", subcore_axis_name="s")

    @functools.partial(
        pl.kernel,
        mesh=mesh,
        out_type=jax.ShapeDtypeStruct((BATCH * DIM,), jnp.float32),
        scratch_types=[
            pltpu.VMEM((IDX_PER_BLK,), jnp.int32),
            pltpu.VMEM((IDX_PER_BLK,), jnp.int32),
            pltpu.VMEM((IDX_PER_BLK, DIM), jnp.float32),
            pltpu.VMEM((IDX_PER_BLK, DIM), jnp.float32),
            pltpu.VMEM((NB * DIM,), jnp.float32),
            pltpu.SemaphoreType.DMA,
            pltpu.SemaphoreType.DMA,
        ],
        compiler_params=pltpu.CompilerParams(use_tc_tiling_on_sc=False),
    )
    def pool_kernel(ids_hbm, table_hbm, out_hbm,
                    idx_v0, idx_v1, rows_v0, rows_v1, out_v, sem0, sem1):
        idx_v = (idx_v0, idx_v1)
        rows_v = (rows_v0, rows_v1)
        sems = (sem0, sem1)
        wid = lax.axis_index("s") * NC + lax.axis_index("c")
        base_row = wid * B_PER_W

        def start(i, buf):
            row0 = base_row + i * NB
            pltpu.sync_copy(ids_hbm.at[pl.ds(row0 * HIST, IDX_PER_BLK)],
                            idx_v[buf])
            pltpu.async_copy(table_hbm.at[idx_v[buf]], rows_v[buf], sems[buf])

        def process(i, buf):
            pltpu.make_async_copy(table_hbm.at[idx_v[buf]], rows_v[buf],
                                  sems[buf]).wait()

            def row_body(b, carry2):
                def inner(r10, accs):
                    accs = list(accs)
                    for k in range(RUNROLL):
                        p = b * HIST + r10 * RUNROLL + k
                        for c in range(NCHUNK):
                            accs[c] = accs[c] + rows_v[buf][p, pl.ds(c * 16, 16)]
                    return tuple(accs)

                accs = lax.fori_loop(
                    0, HIST // RUNROLL, inner,
                    tuple(jnp.zeros((16,), jnp.float32) for _ in range(NCHUNK)),
                )
                for c in range(NCHUNK):
                    out_v[pl.ds(b * DIM + c * 16, 16)] = accs[c]
                return carry2

            lax.fori_loop(0, NB, row_body, 0)
            pltpu.sync_copy(out_v,
                            out_hbm.at[pl.ds((base_row + i * NB) * DIM,
                                             NB * DIM)])

        start(0, 0)

        def pair_body(j, carry):
            i = j * 2
            start(i + 1, 1)
            process(i, 0)

            @pl.when(i + 2 < N_BLK)
            def _():
                start(i + 2, 0)

            process(i + 1, 1)
            return carry

        lax.fori_loop(0, N_BLK // 2, pair_body, 0)

    return pool_kernel(ids_flat, ptab_flat)


def kernel(input_ids, embedding, proj_w, proj_b):
    ids = input_ids.astype(jnp.int32)
    t = jnp.bitwise_and(ids, 2 * VBP - 1)
    ids_flat = (ids + t - jnp.where(t < VBP, 0, 2 * VBP - 1)).reshape(-1)
    ptab = _tc_proj_table(embedding.T, proj_w * (1.0 / HIST),
                          proj_b * (1.0 / HIST))
    ptab_flat = ptab.reshape(TABROWS, DIM)
    out_flat = _sc_pool_sum(ids_flat, ptab_flat)
    return out_flat.reshape(BATCH, DIM)


# speedup vs baseline: 1.0052x; 1.0052x over previous
import functools

import jax
import jax.numpy as jnp
from jax import lax
from jax.experimental import pallas as pl
from jax.experimental.pallas import tpu as pltpu
from jax.experimental.pallas import tpu_sc as plsc

VOCAB = 1000000
DIM = 64
BATCH = 16384
HIST = 50
PADW = 2 * DIM

VBP = 16384
NPAIR = -(-((VOCAB + VBP - 1) // VBP) // 2) * VBP
NGRID = NPAIR // VBP
TABROWS = 2 * NPAIR

NC = 2
NS = 16
NW = NC * NS
B_PER_W = BATCH // NW
NB = 16
IDX_PER_BLK = NB * HIST
N_BLK = B_PER_W // NB
NCHUNK = DIM // 16
RUNROLL = 10


def _tc_proj_table(emb_t, proj_w, proj_b):
    def proj_kernel(e_ref, w_ref, b_ref, o_ref):
        for half in (0, 1):
            p = lax.dot_general(
                e_ref[:, half * VBP:(half + 1) * VBP].T, w_ref[...],
                (((1,), (1,)), ((), ())),
                preferred_element_type=jnp.float32,
            )
            o_ref[:, half * DIM:(half + 1) * DIM] = p + b_ref[...]

    return pl.pallas_call(
        proj_kernel,
        grid=(NGRID,),
        in_specs=[
            pl.BlockSpec((DIM, 2 * VBP), lambda k: (0, k)),
            pl.BlockSpec((DIM, DIM), lambda k: (0, 0)),
            pl.BlockSpec((1, DIM), lambda k: (0, 0)),
        ],
        out_specs=pl.BlockSpec((VBP, PADW), lambda k: (k, 0)),
        out_shape=jax.ShapeDtypeStruct((NPAIR, PADW), jnp.float32),
        compiler_params=pltpu.CompilerParams(
            fuse_transposed_lhs_in_matmul=True),
    )(emb_t, proj_w, proj_b.reshape(1, DIM))


def _sc_pool_sum(ids_flat, ptab_flat):
    mesh = plsc.VectorSubcoreMesh(core_axis_name="c", subcore_axis_name="s")

    @functools.partial(
        pl.kernel,
        mesh=mesh,
        out_type=jax.ShapeDtypeStruct((BATCH * DIM,), jnp.float32),
        scratch_types=[
            pltpu.VMEM((IDX_PER_BLK,), jnp.int32),
            pltpu.VMEM((IDX_PER_BLK,), jnp.int32),
            pltpu.VMEM((IDX_PER_BLK, DIM), jnp.float32),
            pltpu.VMEM((IDX_PER_BLK, DIM), jnp.float32),
            pltpu.VMEM((NB * DIM,), jnp.float32),
            pltpu.SemaphoreType.DMA,
            pltpu.SemaphoreType.DMA,
        ],
        compiler_params=pltpu.CompilerParams(use_tc_tiling_on_sc=False),
    )
    def pool_kernel(ids_hbm, table_hbm, out_hbm,
                    idx_v0, idx_v1, rows_v0, rows_v1, out_v, sem0, sem1):
        idx_v = (idx_v0, idx_v1)
        rows_v = (rows_v0, rows_v1)
        sems = (sem0, sem1)
        wid = lax.axis_index("s") * NC + lax.axis_index("c")
        base_row = wid * B_PER_W

        def start(i, buf):
            row0 = base_row + i * NB
            pltpu.sync_copy(ids_hbm.at[pl.ds(row0 * HIST, IDX_PER_BLK)],
                            idx_v[buf])
            pltpu.async_copy(table_hbm.at[idx_v[buf]], rows_v[buf], sems[buf])

        def process(i, buf):
            pltpu.make_async_copy(table_hbm.at[idx_v[buf]], rows_v[buf],
                                  sems[buf]).wait()

            def row_body(b, carry2):
                def inner(r10, accs):
                    accs = list(accs)
                    for k in range(RUNROLL):
                        p = b * HIST + r10 * RUNROLL + k
                        for c in range(NCHUNK):
                            accs[c] = accs[c] + rows_v[buf][p, pl.ds(c * 16, 16)]
                    return tuple(accs)

                accs = lax.fori_loop(
                    0, HIST // RUNROLL, inner,
                    tuple(jnp.zeros((16,), jnp.float32) for _ in range(NCHUNK)),
                )
                for c in range(NCHUNK):
                    out_v[pl.ds(b * DIM + c * 16, 16)] = accs[c]
                return carry2

            lax.fori_loop(0, NB, row_body, 0)
            pltpu.sync_copy(out_v,
                            out_hbm.at[pl.ds((base_row + i * NB) * DIM,
                                             NB * DIM)])

        start(0, 0)

        def pair_body(j, carry):
            i = j * 2
            start(i + 1, 1)
            process(i, 0)

            @pl.when(i + 2 < N_BLK)
            def _():
                start(i + 2, 0)

            process(i + 1, 1)
            return carry

        lax.fori_loop(0, N_BLK // 2, pair_body, 0)

    return pool_kernel(ids_flat, ptab_flat)


def kernel(input_ids, embedding, proj_w, proj_b):
    ids = input_ids.astype(jnp.int32)
    t = jnp.bitwise_and(ids, 2 * VBP - 1)
    ids_flat = (ids + t - jnp.where(t < VBP, 0, 2 * VBP - 1)).reshape(-1)
    ptab = _tc_proj_table(embedding.T, proj_w * (1.0 / HIST),
                          proj_b * (1.0 / HIST))
    ptab_flat = ptab.reshape(TABROWS, DIM)
    out_flat = _sc_pool_sum(ids_flat, ptab_flat)
    return out_flat.reshape(BATCH, DIM)


# speedup vs baseline: 1.1167x; 1.1109x over previous
import functools

import jax
import jax.numpy as jnp
from jax import lax
from jax.experimental import pallas as pl
from jax.experimental.pallas import tpu as pltpu
from jax.experimental.pallas import tpu_sc as plsc

VOCAB = 1000000
DIM = 64
BATCH = 16384
HIST = 50
PADW = 2 * DIM

VBP = 16384
NPAIR = -(-((VOCAB + VBP - 1) // VBP) // 2) * VBP
NGRID = NPAIR // VBP
TABROWS = 2 * NPAIR

NC = 2
NS = 16
NW = NC * NS
B_PER_W = BATCH // NW
NB = 16
IDX_PER_BLK = NB * HIST
N_BLK = B_PER_W // NB
NCHUNK = DIM // 16
RUNROLL = 10


def _tc_proj_table(emb_t, proj_w, proj_b):
    def proj_kernel(e_ref, w_ref, b_ref, o_ref):
        for half in (0, 1):
            p = lax.dot_general(
                e_ref[:, half * VBP:(half + 1) * VBP].astype(jnp.bfloat16).T,
                w_ref[...].astype(jnp.bfloat16),
                (((1,), (1,)), ((), ())),
                preferred_element_type=jnp.float32,
            )
            o_ref[:, half * DIM:(half + 1) * DIM] = p + b_ref[...]

    return pl.pallas_call(
        proj_kernel,
        grid=(NGRID,),
        in_specs=[
            pl.BlockSpec((DIM, 2 * VBP), lambda k: (0, k)),
            pl.BlockSpec((DIM, DIM), lambda k: (0, 0)),
            pl.BlockSpec((1, DIM), lambda k: (0, 0)),
        ],
        out_specs=pl.BlockSpec((VBP, PADW), lambda k: (k, 0)),
        out_shape=jax.ShapeDtypeStruct((NPAIR, PADW), jnp.float32),
        compiler_params=pltpu.CompilerParams(
            fuse_transposed_lhs_in_matmul=True),
    )(emb_t, proj_w, proj_b.reshape(1, DIM))


def _sc_pool_sum(ids_flat, ptab_flat):
    mesh = plsc.VectorSubcoreMesh(core_axis_name="c", subcore_axis_name="s")

    @functools.partial(
        pl.kernel,
        mesh=mesh,
        out_type=jax.ShapeDtypeStruct((BATCH * DIM,), jnp.float32),
        scratch_types=[
            pltpu.VMEM((IDX_PER_BLK,), jnp.int32),
            pltpu.VMEM((IDX_PER_BLK,), jnp.int32),
            pltpu.VMEM((IDX_PER_BLK, DIM), jnp.float32),
            pltpu.VMEM((IDX_PER_BLK, DIM), jnp.float32),
            pltpu.VMEM((NB * DIM,), jnp.float32),
            pltpu.SemaphoreType.DMA,
            pltpu.SemaphoreType.DMA,
        ],
        compiler_params=pltpu.CompilerParams(use_tc_tiling_on_sc=False),
    )
    def pool_kernel(ids_hbm, table_hbm, out_hbm,
                    idx_v0, idx_v1, rows_v0, rows_v1, out_v, sem0, sem1):
        idx_v = (idx_v0, idx_v1)
        rows_v = (rows_v0, rows_v1)
        sems = (sem0, sem1)
        wid = lax.axis_index("s") * NC + lax.axis_index("c")
        base_row = wid * B_PER_W

        def start(i, buf):
            row0 = base_row + i * NB
            pltpu.sync_copy(ids_hbm.at[pl.ds(row0 * HIST, IDX_PER_BLK)],
                            idx_v[buf])
            pltpu.async_copy(table_hbm.at[idx_v[buf]], rows_v[buf], sems[buf])

        def process(i, buf):
            pltpu.make_async_copy(table_hbm.at[idx_v[buf]], rows_v[buf],
                                  sems[buf]).wait()

            def row_body(b, carry2):
                def inner(r10, accs):
                    accs = list(accs)
                    for k in range(RUNROLL):
                        p = b * HIST + r10 * RUNROLL + k
                        for c in range(NCHUNK):
                            accs[c] = accs[c] + rows_v[buf][p, pl.ds(c * 16, 16)]
                    return tuple(accs)

                accs = lax.fori_loop(
                    0, HIST // RUNROLL, inner,
                    tuple(jnp.zeros((16,), jnp.float32) for _ in range(NCHUNK)),
                )
                for c in range(NCHUNK):
                    out_v[pl.ds(b * DIM + c * 16, 16)] = accs[c]
                return carry2

            lax.fori_loop(0, NB, row_body, 0)
            pltpu.sync_copy(out_v,
                            out_hbm.at[pl.ds((base_row + i * NB) * DIM,
                                             NB * DIM)])

        start(0, 0)

        def pair_body(j, carry):
            i = j * 2
            start(i + 1, 1)
            process(i, 0)

            @pl.when(i + 2 < N_BLK)
            def _():
                start(i + 2, 0)

            process(i + 1, 1)
            return carry

        lax.fori_loop(0, N_BLK // 2, pair_body, 0)

    return pool_kernel(ids_flat, ptab_flat)


def kernel(input_ids, embedding, proj_w, proj_b):
    ids = input_ids.astype(jnp.int32)
    t = jnp.bitwise_and(ids, 2 * VBP - 1)
    ids_flat = (ids + t - jnp.where(t < VBP, 0, 2 * VBP - 1)).reshape(-1)
    ptab = _tc_proj_table(embedding.T, proj_w * (1.0 / HIST),
                          proj_b * (1.0 / HIST))
    ptab_flat = ptab.reshape(TABROWS, DIM)
    out_flat = _sc_pool_sum(ids_flat, ptab_flat)
    return out_flat.reshape(BATCH, DIM)


# speedup vs baseline: 1.1526x; 1.0321x over previous
import functools

import jax
import jax.numpy as jnp
from jax import lax
from jax.experimental import pallas as pl
from jax.experimental.pallas import tpu as pltpu
from jax.experimental.pallas import tpu_sc as plsc

VOCAB = 1000000
DIM = 64
BATCH = 16384
HIST = 50
PADW = 2 * DIM

VBP = 16384
NPAIR = -(-((VOCAB + VBP - 1) // VBP) // 2) * VBP
NGRID = NPAIR // VBP
TABROWS = 2 * NPAIR

NC = 2
NS = 16
NW = NC * NS
B_PER_W = BATCH // NW
NB = 16
IDX_PER_BLK = NB * HIST
N_BLK = B_PER_W // NB
NCHUNK = DIM // 16
RUNROLL = 10


def _tc_proj_table(emb_t, proj_w, proj_b):
    def proj_kernel(e_ref, w_ref, b_ref, o_ref):
        eb = e_ref[...].astype(jnp.bfloat16)
        lhs = jnp.concatenate([eb[:, :VBP].T, eb[:, VBP:].T], axis=1)
        p = lax.dot_general(
            lhs, w_ref[...],
            (((1,), (0,)), ((), ())),
            preferred_element_type=jnp.float32,
        )
        o_ref[...] = p + b_ref[...]

    return pl.pallas_call(
        proj_kernel,
        grid=(NGRID,),
        in_specs=[
            pl.BlockSpec((DIM, 2 * VBP), lambda k: (0, k)),
            pl.BlockSpec((PADW, PADW), lambda k: (0, 0)),
            pl.BlockSpec((1, PADW), lambda k: (0, 0)),
        ],
        out_specs=pl.BlockSpec((VBP, PADW), lambda k: (k, 0)),
        out_shape=jax.ShapeDtypeStruct((NPAIR, PADW), jnp.float32),
        compiler_params=pltpu.CompilerParams(
            fuse_transposed_lhs_in_matmul=True),
    )(emb_t, proj_w, proj_b.reshape(1, PADW))


def _sc_pool_sum(ids_flat, ptab_flat):
    mesh = plsc.VectorSubcoreMesh(core_axis_name="c", subcore_axis_name="s")

    @functools.partial(
        pl.kernel,
        mesh=mesh,
        out_type=jax.ShapeDtypeStruct((BATCH * DIM,), jnp.float32),
        scratch_types=[
            pltpu.VMEM((IDX_PER_BLK,), jnp.int32),
            pltpu.VMEM((IDX_PER_BLK,), jnp.int32),
            pltpu.VMEM((IDX_PER_BLK, DIM), jnp.float32),
            pltpu.VMEM((IDX_PER_BLK, DIM), jnp.float32),
            pltpu.VMEM((NB * DIM,), jnp.float32),
            pltpu.SemaphoreType.DMA,
            pltpu.SemaphoreType.DMA,
        ],
        compiler_params=pltpu.CompilerParams(use_tc_tiling_on_sc=False),
    )
    def pool_kernel(ids_hbm, table_hbm, out_hbm,
                    idx_v0, idx_v1, rows_v0, rows_v1, out_v, sem0, sem1):
        idx_v = (idx_v0, idx_v1)
        rows_v = (rows_v0, rows_v1)
        sems = (sem0, sem1)
        wid = lax.axis_index("s") * NC + lax.axis_index("c")
        base_row = wid * B_PER_W

        def start(i, buf):
            row0 = base_row + i * NB
            pltpu.sync_copy(ids_hbm.at[pl.ds(row0 * HIST, IDX_PER_BLK)],
                            idx_v[buf])
            pltpu.async_copy(table_hbm.at[idx_v[buf]], rows_v[buf], sems[buf])

        def process(i, buf):
            pltpu.make_async_copy(table_hbm.at[idx_v[buf]], rows_v[buf],
                                  sems[buf]).wait()

            def row_body(b, carry2):
                def inner(r10, accs):
                    accs = list(accs)
                    for k in range(RUNROLL):
                        p = b * HIST + r10 * RUNROLL + k
                        for c in range(NCHUNK):
                            accs[c] = accs[c] + rows_v[buf][p, pl.ds(c * 16, 16)]
                    return tuple(accs)

                accs = lax.fori_loop(
                    0, HIST // RUNROLL, inner,
                    tuple(jnp.zeros((16,), jnp.float32) for _ in range(NCHUNK)),
                )
                for c in range(NCHUNK):
                    out_v[pl.ds(b * DIM + c * 16, 16)] = accs[c]
                return carry2

            lax.fori_loop(0, NB, row_body, 0)
            pltpu.sync_copy(out_v,
                            out_hbm.at[pl.ds((base_row + i * NB) * DIM,
                                             NB * DIM)])

        start(0, 0)

        def pair_body(j, carry):
            i = j * 2
            start(i + 1, 1)
            process(i, 0)

            @pl.when(i + 2 < N_BLK)
            def _():
                start(i + 2, 0)

            process(i + 1, 1)
            return carry

        lax.fori_loop(0, N_BLK // 2, pair_body, 0)

    return pool_kernel(ids_flat, ptab_flat)


def kernel(input_ids, embedding, proj_w, proj_b):
    ids = input_ids.astype(jnp.int32)
    t = jnp.bitwise_and(ids, 2 * VBP - 1)
    ids_flat = (ids + t - jnp.where(t < VBP, 0, 2 * VBP - 1)).reshape(-1)
    wt = (proj_w * (1.0 / HIST)).T.astype(jnp.bfloat16)
    z = jnp.zeros((DIM, DIM), jnp.bfloat16)
    w2 = jnp.block([[wt, z], [z, wt]])
    b2 = jnp.tile(proj_b * (1.0 / HIST), 2)
    ptab = _tc_proj_table(embedding.T, w2, b2)
    ptab_flat = ptab.reshape(TABROWS, DIM)
    out_flat = _sc_pool_sum(ids_flat, ptab_flat)
    return out_flat.reshape(BATCH, DIM)
